# Initial kernel scaffold; baseline (speedup 1.0000x reference)
import functools

import jax
import jax.numpy as jnp
from jax import lax
from jax.experimental import pallas as pl
from jax.experimental.pallas import tpu as pltpu
from jax.experimental.pallas import tpu_sc as plsc

N_USERS = 5000
N_ITEMS = 5000
N_NODES = 10000
NT = 2 * N_NODES
D = 128
E = 320000
ALPHA = 0.5

NUM_TILES = 16
CHUNK = 128
NCH = 160
EPT = NCH * CHUNK
E_PAD = NUM_TILES * EPT
ROWS_PT = N_NODES // NUM_TILES
INIT_SUB = 125


def _scale_rows(rows_ref, nrows, get_w):
  def body(r, _):
    wr = get_w(r)
    for t in range(D // 16):
      sl = pl.ds(t * 16, 16)
      rows_ref[r, sl] = rows_ref[r, sl] * wr
    return 0
  lax.fori_loop(0, nrows, body, 0)


def _sc_layer_kernel(x_hbm, src_hbm, dst_hbm, w_hbm, out_hbm,
                     src_v, dst_v, w_v, rows0, rows1, acc, gsem0, gsem1):
  c = lax.axis_index("c")
  s = lax.axis_index("s")

  row0 = s * ROWS_PT
  for t in range(ROWS_PT // INIT_SUB):
    r = row0 + t * INIT_SUB
    pltpu.sync_copy(x_hbm.at[pl.ds(c * N_NODES + r, INIT_SUB)],
                    rows0.at[pl.ds(0, INIT_SUB)])
    _scale_rows(rows0, INIT_SUB, lambda _: ALPHA)
    pltpu.sync_copy(rows0.at[pl.ds(0, INIT_SUB)], acc.at[pl.ds(r, INIT_SUB)])
  plsc.subcore_barrier()

  tb = s * NCH
  pltpu.sync_copy(src_hbm.at[c, pl.ds(tb, NCH)], src_v)
  pltpu.sync_copy(dst_hbm.at[c, pl.ds(tb, NCH)], dst_v)
  pltpu.sync_copy(w_hbm.at[c, pl.ds(tb, NCH)], w_v)

  bufs = (rows0, rows1)
  sems = (gsem0, gsem1)
  pltpu.async_copy(x_hbm.at[src_v.at[0]], rows0, gsem0)

  def chunk_pair(i, _):
    jb = i * 2
    for b in range(2):
      j = jb + b
      rows, gsem = bufs[b], sems[b]
      nrows, nsem = bufs[1 - b], sems[1 - b]
      pltpu.make_async_copy(x_hbm.at[src_v.at[j]], rows, gsem).wait()

      @pl.when(j + 1 < NCH)
      def _():
        pltpu.async_copy(x_hbm.at[src_v.at[j + 1]], nrows, nsem)

      _scale_rows(rows, CHUNK, lambda r: w_v[j, r])
      pltpu.sync_copy(rows, acc.at[dst_v.at[j]], add=True)
    return 0

  lax.fori_loop(0, NCH // 2, chunk_pair, 0)

  plsc.subcore_barrier()
  pltpu.sync_copy(acc.at[pl.ds(row0, ROWS_PT)],
                  out_hbm.at[pl.ds(c * N_NODES + row0, ROWS_PT)])


def _sc_layer(x, src, dst, w):
  mesh = plsc.VectorSubcoreMesh(core_axis_name="c", subcore_axis_name="s")
  fn = functools.partial(
      pl.kernel,
      out_type=jax.ShapeDtypeStruct((NT, D), jnp.float32),
      mesh=mesh,
      scratch_types=[
          pltpu.VMEM((NCH, CHUNK), jnp.int32),
          pltpu.VMEM((NCH, CHUNK), jnp.int32),
          pltpu.VMEM((NCH, CHUNK), jnp.float32),
          pltpu.VMEM((CHUNK, D), jnp.float32),
          pltpu.VMEM((CHUNK, D), jnp.float32),
          pltpu.VMEM_SHARED((N_NODES, D), jnp.float32),
          pltpu.SemaphoreType.DMA,
          pltpu.SemaphoreType.DMA,
      ],
  )(_sc_layer_kernel)
  return fn(x, src, dst, w)


def _project_norm(feats, W, b):
  n_rows, K = feats.shape
  R = 500

  def body(f_ref, w_ref, b_ref, o_ref):
    z = jnp.dot(f_ref[...], w_ref[...],
                preferred_element_type=jnp.float32) + b_ref[...]
    n = jnp.sqrt(jnp.sum(z * z, axis=1, keepdims=True))
    o_ref[...] = z / jnp.maximum(n, 1e-12)

  return pl.pallas_call(
      body,
      grid=(n_rows // R,),
      in_specs=[
          pl.BlockSpec((R, K), lambda i: (i, 0)),
          pl.BlockSpec((K, D), lambda i: (0, 0)),
          pl.BlockSpec((1, D), lambda i: (0, 0)),
      ],
      out_specs=pl.BlockSpec((R, D), lambda i: (i, 0)),
      out_shape=jax.ShapeDtypeStruct((n_rows, D), jnp.float32),
  )(feats, W, b.reshape(1, D))


def _mix(x2, modal_weight):
  def body(x_ref, mw_ref, u_ref, i_ref):
    mw = mw_ref[...]
    e = jnp.exp(mw - jnp.max(mw))
    wgt = e / jnp.sum(e)
    w0, w1 = wgt[0, 0], wgt[0, 1]
    u_ref[...] = (w0 * x_ref[:N_USERS, :]
                  + w1 * x_ref[N_NODES:N_NODES + N_USERS, :])
    i_ref[...] = (w0 * x_ref[N_USERS:N_NODES, :]
                  + w1 * x_ref[N_NODES + N_USERS:, :])

  return pl.pallas_call(
      body,
      out_shape=(jax.ShapeDtypeStruct((N_USERS, D), jnp.float32),
                 jax.ShapeDtypeStruct((N_ITEMS, D), jnp.float32)),
  )(x2, modal_weight.reshape(1, 2))


def _prep_edges(edge_index, edge_weight, src_offset):
  pad = E_PAD - E
  src = jnp.pad(edge_index[0], (0, pad)) + src_offset
  dst = jnp.pad(edge_index[1], (0, pad))
  wv = jnp.pad(edge_weight[:, 0], (0, pad))
  shape = (NUM_TILES * NCH, CHUNK)
  return src.reshape(shape), dst.reshape(shape), wv.reshape(shape)


def kernel(edge_index_img, edge_weight_img, edge_index_txt, edge_weight_txt,
           image_feats, text_feats, W_img, b_img, W_txt, b_txt,
           image_preference, text_preference, modal_weight):
  img_emb = _project_norm(image_feats, W_img, b_img)
  txt_emb = _project_norm(text_feats, W_txt, b_txt)
  x = jnp.concatenate(
      [image_preference, img_emb, text_preference, txt_emb], axis=0)

  si, di, wi = _prep_edges(edge_index_img, edge_weight_img, 0)
  st, dt, wt = _prep_edges(edge_index_txt, edge_weight_txt, N_NODES)
  src = jnp.stack([si, st])
  dst = jnp.stack([di, dt])
  w = jnp.stack([wi, wt])

  for _ in range(2):
    x = _sc_layer(x, src, dst, w)

  return _mix(x, modal_weight)


# speedup vs baseline: 3.2177x; 3.2177x over previous
import functools

import jax
import jax.numpy as jnp
from jax import lax
from jax.experimental import pallas as pl
from jax.experimental.pallas import tpu as pltpu
from jax.experimental.pallas import tpu_sc as plsc

N_USERS = 5000
N_ITEMS = 5000
N_NODES = 10000
NP = 10240
NT = 2 * NP
D = 128
E = 320000
ALPHA = 0.5

NUM_TILES = 16
CHUNK = 128
NCH = 160
EPT = NCH * CHUNK
E_PAD = NUM_TILES * EPT
ROWS_PT = NP // NUM_TILES
INIT_SUB = 128


def _scale_rows_const(rows_ref, nrows, cval):
  def body(r, _):
    for t in range(D // 16):
      sl = pl.ds(t * 16, 16)
      rows_ref[r, sl] = rows_ref[r, sl] * cval
    return 0
  lax.fori_loop(0, nrows, body, 0)


def _scale_rows_by_weights(rows_ref, w_ref, j):
  def body(g, _):
    wvec = w_ref[j, pl.ds(g * 16, 16)]
    for lane in range(16):
      wr = wvec[lane]
      r = g * 16 + lane
      for t in range(D // 16):
        sl = pl.ds(t * 16, 16)
        rows_ref[r, sl] = rows_ref[r, sl] * wr
    return 0
  lax.fori_loop(0, CHUNK // 16, body, 0)


def _sc_layer_kernel(x_hbm, src_hbm, dst_hbm, w_hbm, out_hbm,
                     src_c, dst_c, w_c, rows0, rows1, acc,
                     gsem0, gsem1, isem0, isem1):
  c = lax.axis_index("c")
  s = lax.axis_index("s")

  row0 = s * ROWS_PT
  for t in range(ROWS_PT // INIT_SUB):
    r = row0 + t * INIT_SUB
    pltpu.sync_copy(x_hbm.at[pl.ds(c * NP + r, INIT_SUB)],
                    rows0.at[pl.ds(0, INIT_SUB)])
    _scale_rows_const(rows0, INIT_SUB, ALPHA)
    pltpu.sync_copy(rows0.at[pl.ds(0, INIT_SUB)], acc.at[pl.ds(r, INIT_SUB)])
  plsc.subcore_barrier()

  tb = s * NCH
  bufs = (rows0, rows1)
  gsems = (gsem0, gsem1)
  isems = (isem0, isem1)

  def idx_copies(j, slot):
    sem = isems[slot]
    return (
        pltpu.make_async_copy(src_hbm.at[c, tb + j], src_c.at[slot], sem),
        pltpu.make_async_copy(dst_hbm.at[c, tb + j], dst_c.at[slot], sem),
        pltpu.make_async_copy(w_hbm.at[c, tb + j], w_c.at[slot], sem),
    )

  def idx_start(j, slot):
    for d in idx_copies(j, slot):
      d.start()

  def idx_wait(j, slot):
    for d in idx_copies(j, slot):
      d.wait()

  def gather(j_slot, buf, gsem):
    return pltpu.make_async_copy(x_hbm.at[src_c.at[j_slot]], buf, gsem)

  idx_start(0, 0)
  idx_wait(0, 0)
  gather(0, rows0, gsem0).start()
  idx_start(1, 1)

  def chunk_pair(i, _):
    jb = i * 2
    for b in range(2):
      j = jb + b
      nb = 1 - b

      @pl.when(j + 1 < NCH)
      def _():
        idx_wait(j + 1, nb)
        gather(nb, bufs[nb], gsems[nb]).start()

      gather(b, bufs[b], gsems[b]).wait()
      _scale_rows_by_weights(bufs[b], w_c, b)
      pltpu.sync_copy(bufs[b], acc.at[dst_c.at[b]], add=True)

      @pl.when(j + 2 < NCH)
      def _():
        idx_start(j + 2, b)
    return 0

  lax.fori_loop(0, NCH // 2, chunk_pair, 0)

  plsc.subcore_barrier()
  pltpu.sync_copy(acc.at[pl.ds(row0, ROWS_PT)],
                  out_hbm.at[pl.ds(c * NP + row0, ROWS_PT)])


def _sc_layer(x, src, dst, w):
  mesh = plsc.VectorSubcoreMesh(core_axis_name="c", subcore_axis_name="s")
  fn = functools.partial(
      pl.kernel,
      out_type=jax.ShapeDtypeStruct((NT, D), jnp.float32),
      mesh=mesh,
      scratch_types=[
          pltpu.VMEM((2, CHUNK), jnp.int32),
          pltpu.VMEM((2, CHUNK), jnp.int32),
          pltpu.VMEM((2, CHUNK), jnp.float32),
          pltpu.VMEM((CHUNK, D), jnp.float32),
          pltpu.VMEM((CHUNK, D), jnp.float32),
          pltpu.VMEM_SHARED((NP, D), jnp.float32),
          pltpu.SemaphoreType.DMA,
          pltpu.SemaphoreType.DMA,
          pltpu.SemaphoreType.DMA,
          pltpu.SemaphoreType.DMA,
      ],
  )(_sc_layer_kernel)
  return fn(x, src, dst, w)


def _project_norm(feats, W, b):
  n_rows, K = feats.shape
  R = 1000

  def body(f_ref, w_ref, b_ref, o_ref):
    z = jnp.dot(f_ref[...], w_ref[...],
                preferred_element_type=jnp.float32) + b_ref[...]
    n = jnp.sqrt(jnp.sum(z * z, axis=1, keepdims=True))
    o_ref[...] = z / jnp.maximum(n, 1e-12)

  return pl.pallas_call(
      body,
      grid=(n_rows // R,),
      in_specs=[
          pl.BlockSpec((R, K), lambda i: (i, 0)),
          pl.BlockSpec((K, D), lambda i: (0, 0)),
          pl.BlockSpec((1, D), lambda i: (0, 0)),
      ],
      out_specs=pl.BlockSpec((R, D), lambda i: (i, 0)),
      out_shape=jax.ShapeDtypeStruct((n_rows, D), jnp.float32),
  )(feats, W, b.reshape(1, D))


def _mix(x2, modal_weight):
  def body(x_ref, mw_ref, u_ref, i_ref):
    mw = mw_ref[...]
    e = jnp.exp(mw - jnp.max(mw))
    wgt = e / jnp.sum(e)
    w0, w1 = wgt[0, 0], wgt[0, 1]
    u_ref[...] = (w0 * x_ref[:N_USERS, :]
                  + w1 * x_ref[NP:NP + N_USERS, :])
    i_ref[...] = (w0 * x_ref[N_USERS:N_NODES, :]
                  + w1 * x_ref[NP + N_USERS:NP + N_NODES, :])

  return pl.pallas_call(
      body,
      out_shape=(jax.ShapeDtypeStruct((N_USERS, D), jnp.float32),
                 jax.ShapeDtypeStruct((N_ITEMS, D), jnp.float32)),
  )(x2, modal_weight.reshape(1, 2))


def _prep_edges(edge_index, edge_weight, src_offset):
  pad = E_PAD - E
  src = jnp.pad(edge_index[0], (0, pad)) + src_offset
  dst = jnp.pad(edge_index[1], (0, pad))
  wv = jnp.pad(edge_weight[:, 0], (0, pad))
  shape = (NUM_TILES * NCH, CHUNK)
  return src.reshape(shape), dst.reshape(shape), wv.reshape(shape)


def kernel(edge_index_img, edge_weight_img, edge_index_txt, edge_weight_txt,
           image_feats, text_feats, W_img, b_img, W_txt, b_txt,
           image_preference, text_preference, modal_weight):
  img_emb = _project_norm(image_feats, W_img, b_img)
  txt_emb = _project_norm(text_feats, W_txt, b_txt)
  zpad = jnp.zeros((NP - N_NODES, D), jnp.float32)
  x = jnp.concatenate(
      [image_preference, img_emb, zpad, text_preference, txt_emb, zpad],
      axis=0)

  si, di, wi = _prep_edges(edge_index_img, edge_weight_img, 0)
  st, dt, wt = _prep_edges(edge_index_txt, edge_weight_txt, NP)
  src = jnp.stack([si, st])
  dst = jnp.stack([di, dt])
  w = jnp.stack([wi, wt])

  for _ in range(2):
    x = _sc_layer(x, src, dst, w)

  return _mix(x, modal_weight)


# speedup vs baseline: 3.2266x; 1.0028x over previous
import functools

import jax
import jax.numpy as jnp
from jax import lax
from jax.experimental import pallas as pl
from jax.experimental.pallas import tpu as pltpu
from jax.experimental.pallas import tpu_sc as plsc

N_USERS = 5000
N_ITEMS = 5000
N_NODES = 10000
NP = 10240
NT = 2 * NP
D = 128
E = 320000
ALPHA = 0.5

NUM_TILES = 16
CHUNK = 128
NCH = 160
EPT = NCH * CHUNK
E_PAD = NUM_TILES * EPT
ROWS_PT = NP // NUM_TILES
INIT_SUB = 128


def _scale_rows_const(rows_ref, nrows, cval):
  @plsc.parallel_loop(0, nrows, unroll=4)
  def _(r):
    for t in range(D // 16):
      sl = pl.ds(t * 16, 16)
      rows_ref[r, sl] = rows_ref[r, sl] * cval


def _scale_rows_by_weights(rows_ref, w_ref, j):
  @plsc.parallel_loop(0, CHUNK // 16, unroll=2)
  def _(g):
    wvec = w_ref[j, pl.ds(g * 16, 16)]
    for lane in range(16):
      wr = wvec[lane]
      r = g * 16 + lane
      for t in range(D // 16):
        sl = pl.ds(t * 16, 16)
        rows_ref[r, sl] = rows_ref[r, sl] * wr


def _sc_layer_kernel(x_hbm, src_hbm, dst_hbm, w_hbm, out_hbm,
                     src_c, dst_c, w_c, rows0, rows1, acc,
                     gsem0, gsem1, isem0, isem1):
  c = lax.axis_index("c")
  s = lax.axis_index("s")

  row0 = s * ROWS_PT
  for t in range(ROWS_PT // INIT_SUB):
    r = row0 + t * INIT_SUB
    pltpu.sync_copy(x_hbm.at[pl.ds(c * NP + r, INIT_SUB)],
                    rows0.at[pl.ds(0, INIT_SUB)])
    _scale_rows_const(rows0, INIT_SUB, ALPHA)
    pltpu.sync_copy(rows0.at[pl.ds(0, INIT_SUB)], acc.at[pl.ds(r, INIT_SUB)])
  plsc.subcore_barrier()

  tb = s * NCH
  bufs = (rows0, rows1)
  gsems = (gsem0, gsem1)
  isems = (isem0, isem1)

  def idx_copies(j, slot):
    sem = isems[slot]
    return (
        pltpu.make_async_copy(src_hbm.at[c, tb + j], src_c.at[slot], sem),
        pltpu.make_async_copy(dst_hbm.at[c, tb + j], dst_c.at[slot], sem),
        pltpu.make_async_copy(w_hbm.at[c, tb + j], w_c.at[slot], sem),
    )

  def idx_start(j, slot):
    for d in idx_copies(j, slot):
      d.start()

  def idx_wait(j, slot):
    for d in idx_copies(j, slot):
      d.wait()

  def gather(j_slot, buf, gsem):
    return pltpu.make_async_copy(x_hbm.at[src_c.at[j_slot]], buf, gsem)

  idx_start(0, 0)
  idx_wait(0, 0)
  gather(0, rows0, gsem0).start()
  idx_start(1, 1)

  def chunk_pair(i, _):
    jb = i * 2
    for b in range(2):
      j = jb + b
      nb = 1 - b

      @pl.when(j + 1 < NCH)
      def _():
        idx_wait(j + 1, nb)
        gather(nb, bufs[nb], gsems[nb]).start()

      gather(b, bufs[b], gsems[b]).wait()
      _scale_rows_by_weights(bufs[b], w_c, b)
      pltpu.sync_copy(bufs[b], acc.at[dst_c.at[b]], add=True)

      @pl.when(j + 2 < NCH)
      def _():
        idx_start(j + 2, b)
    return 0

  lax.fori_loop(0, NCH // 2, chunk_pair, 0)

  plsc.subcore_barrier()
  pltpu.sync_copy(acc.at[pl.ds(row0, ROWS_PT)],
                  out_hbm.at[pl.ds(c * NP + row0, ROWS_PT)])


def _sc_layer(x, src, dst, w):
  mesh = plsc.VectorSubcoreMesh(core_axis_name="c", subcore_axis_name="s")
  fn = functools.partial(
      pl.kernel,
      out_type=jax.ShapeDtypeStruct((NT, D), jnp.float32),
      mesh=mesh,
      scratch_types=[
          pltpu.VMEM((2, CHUNK), jnp.int32),
          pltpu.VMEM((2, CHUNK), jnp.int32),
          pltpu.VMEM((2, CHUNK), jnp.float32),
          pltpu.VMEM((CHUNK, D), jnp.float32),
          pltpu.VMEM((CHUNK, D), jnp.float32),
          pltpu.VMEM_SHARED((NP, D), jnp.float32),
          pltpu.SemaphoreType.DMA,
          pltpu.SemaphoreType.DMA,
          pltpu.SemaphoreType.DMA,
          pltpu.SemaphoreType.DMA,
      ],
  )(_sc_layer_kernel)
  return fn(x, src, dst, w)


def _project_norm(feats, W, b):
  n_rows, K = feats.shape
  R = 1000

  def body(f_ref, w_ref, b_ref, o_ref):
    z = jnp.dot(f_ref[...], w_ref[...],
                preferred_element_type=jnp.float32) + b_ref[...]
    n = jnp.sqrt(jnp.sum(z * z, axis=1, keepdims=True))
    o_ref[...] = z / jnp.maximum(n, 1e-12)

  return pl.pallas_call(
      body,
      grid=(n_rows // R,),
      in_specs=[
          pl.BlockSpec((R, K), lambda i: (i, 0)),
          pl.BlockSpec((K, D), lambda i: (0, 0)),
          pl.BlockSpec((1, D), lambda i: (0, 0)),
      ],
      out_specs=pl.BlockSpec((R, D), lambda i: (i, 0)),
      out_shape=jax.ShapeDtypeStruct((n_rows, D), jnp.float32),
  )(feats, W, b.reshape(1, D))


def _mix(x2, modal_weight):
  def body(x_ref, mw_ref, u_ref, i_ref):
    mw = mw_ref[...]
    e = jnp.exp(mw - jnp.max(mw))
    wgt = e / jnp.sum(e)
    w0, w1 = wgt[0, 0], wgt[0, 1]
    u_ref[...] = (w0 * x_ref[:N_USERS, :]
                  + w1 * x_ref[NP:NP + N_USERS, :])
    i_ref[...] = (w0 * x_ref[N_USERS:N_NODES, :]
                  + w1 * x_ref[NP + N_USERS:NP + N_NODES, :])

  return pl.pallas_call(
      body,
      out_shape=(jax.ShapeDtypeStruct((N_USERS, D), jnp.float32),
                 jax.ShapeDtypeStruct((N_ITEMS, D), jnp.float32)),
  )(x2, modal_weight.reshape(1, 2))


def _prep_edges(edge_index, edge_weight, src_offset):
  pad = E_PAD - E
  src = jnp.pad(edge_index[0], (0, pad)) + src_offset
  dst = jnp.pad(edge_index[1], (0, pad))
  wv = jnp.pad(edge_weight[:, 0], (0, pad))
  shape = (NUM_TILES * NCH, CHUNK)
  return src.reshape(shape), dst.reshape(shape), wv.reshape(shape)


def kernel(edge_index_img, edge_weight_img, edge_index_txt, edge_weight_txt,
           image_feats, text_feats, W_img, b_img, W_txt, b_txt,
           image_preference, text_preference, modal_weight):
  img_emb = _project_norm(image_feats, W_img, b_img)
  txt_emb = _project_norm(text_feats, W_txt, b_txt)
  zpad = jnp.zeros((NP - N_NODES, D), jnp.float32)
  x = jnp.concatenate(
      [image_preference, img_emb, zpad, text_preference, txt_emb, zpad],
      axis=0)

  si, di, wi = _prep_edges(edge_index_img, edge_weight_img, 0)
  st, dt, wt = _prep_edges(edge_index_txt, edge_weight_txt, NP)
  src = jnp.stack([si, st])
  dst = jnp.stack([di, dt])
  w = jnp.stack([wi, wt])

  for _ in range(2):
    x = _sc_layer(x, src, dst, w)

  return _mix(x, modal_weight)
